# baseline (speedup 1.0000x reference)
import functools

import jax
import jax.numpy as jnp
from jax import lax
from jax.experimental import pallas as pl
from jax.experimental.pallas import tpu as pltpu
from jax.experimental.pallas import tpu_sc as plsc

N = 5000
TH = 0.5
L = 16
NS = 16
B = 128
N_PAD = 5120
NB = N_PAD // B
NCB = B // L
C = N_PAD // L
PAD = -1e30


def _nms_body(x1h, y1h, x2h, y2h, oh, outh,
              x1v, y1v, x2v, y2v, arv, alv, odv, stv, shv):
    sid = lax.axis_index("s")

    pltpu.sync_copy(x1h, x1v)
    pltpu.sync_copy(y1h, y1v)
    pltpu.sync_copy(x2h, x2v)
    pltpu.sync_copy(y2h, y2v)
    pltpu.sync_copy(oh, odv)

    def init_c(c, _):
        o = c * L
        w = jnp.maximum(x2v[pl.ds(o, L)] - x1v[pl.ds(o, L)], 0.0)
        h = jnp.maximum(y2v[pl.ds(o, L)] - y1v[pl.ds(o, L)], 0.0)
        arv[pl.ds(o, L)] = w * h
        alv[pl.ds(o, L)] = jnp.full((L,), 1.0, jnp.float32)
        return 0

    lax.fori_loop(0, N_PAD // L, init_c, 0)
    lanes = lax.iota(jnp.int32, L)

    def block_body(k, _):
        base = k * B
        j = jnp.remainder(sid - k * NCB, NS)

        @pl.when(j < NCB)
        def _():
            pltpu.sync_copy(alv.at[pl.ds(base + j * L, L)],
                            shv.at[pl.ds(j * L, L)])

        plsc.subcore_barrier()
        pltpu.sync_copy(shv, alv.at[pl.ds(base, B)])
        plsc.subcore_barrier()

        def i_body(i, _):
            gi = base + i
            ci = gi // L
            li = gi % L
            oi = ci * L
            liv = jnp.full((L,), li, jnp.int32)

            def splat(ref):
                return ref[pl.ds(oi, L)].at[liv].get(mode="promise_in_bounds")

            a_i = alv[pl.ds(gi, L)][0]

            @pl.when(a_i > 0.0)
            def _():
                x1i = splat(x1v)
                y1i = splat(y1v)
                x2i = splat(x2v)
                y2i = splat(y2v)
                ari = splat(arv)

                def sup_chunk(c, extra=None):
                    o = c * L
                    ix1 = jnp.maximum(x1v[pl.ds(o, L)], x1i)
                    iy1 = jnp.maximum(y1v[pl.ds(o, L)], y1i)
                    ix2 = jnp.minimum(x2v[pl.ds(o, L)], x2i)
                    iy2 = jnp.minimum(y2v[pl.ds(o, L)], y2i)
                    inter = (jnp.maximum(ix2 - ix1, 0.0)
                             * jnp.maximum(iy2 - iy1, 0.0))
                    union = arv[pl.ds(o, L)] + ari - inter
                    sup = inter > union * TH
                    if extra is not None:
                        sup = jnp.logical_and(sup, extra)
                    alv[pl.ds(o, L)] = jnp.where(sup, 0.0, alv[pl.ds(o, L)])

                sup_chunk(ci, lanes > li)

                @plsc.parallel_loop(ci + 1, (k + 1) * NCB, unroll=2)
                def _rest(c):
                    sup_chunk(c)

                start = (k + 1) * NCB
                c0 = start + jnp.remainder(sid - start, NS)

                @plsc.parallel_loop(c0, C, step=NS, unroll=8)
                def _tail(c):
                    sup_chunk(c)

            return 0

        hi = jnp.minimum(B, N - base)
        lax.fori_loop(0, hi, i_body, 0)
        return 0

    lax.fori_loop(0, NB, block_body, 0)

    def out_body(m, _):
        o = (m * NS + sid) * L
        stv[...] = jnp.where(alv[pl.ds(o, L)] > 0.0, odv[pl.ds(o, L)],
                             jnp.full((L,), -1, jnp.int32))
        pltpu.sync_copy(stv, outh.at[pl.ds(o, L)])
        return 0

    lax.fori_loop(0, C // NS, out_body, 0)


_nms_sc = functools.partial(
    pl.kernel,
    out_type=jax.ShapeDtypeStruct((N_PAD,), jnp.int32),
    mesh=plsc.VectorSubcoreMesh(core_axis_name="c", subcore_axis_name="s",
                                num_cores=1, num_subcores=NS),
    scratch_types=[
        pltpu.VMEM((N_PAD,), jnp.float32),
        pltpu.VMEM((N_PAD,), jnp.float32),
        pltpu.VMEM((N_PAD,), jnp.float32),
        pltpu.VMEM((N_PAD,), jnp.float32),
        pltpu.VMEM((N_PAD,), jnp.float32),
        pltpu.VMEM((N_PAD + L,), jnp.float32),
        pltpu.VMEM((N_PAD,), jnp.int32),
        pltpu.VMEM((L,), jnp.int32),
        pltpu.VMEM_SHARED((B,), jnp.float32),
    ],
)(_nms_body)


def kernel(boxes, scores, class_ids):
    max_c = boxes.max()
    offs = class_ids.astype(boxes.dtype) * (max_c + 1.0)
    b = boxes + offs[:, None]
    order = jnp.argsort(-scores)
    bs = b[order]
    padc = jnp.full((N_PAD - N,), PAD, jnp.float32)
    x1 = jnp.concatenate([bs[:, 0], padc])
    y1 = jnp.concatenate([bs[:, 1], padc])
    x2 = jnp.concatenate([bs[:, 2], padc])
    y2 = jnp.concatenate([bs[:, 3], padc])
    ordp = jnp.concatenate(
        [order.astype(jnp.int32), jnp.full((N_PAD - N,), -1, jnp.int32)])
    out = _nms_sc(x1, y1, x2, y2, ordp)
    return out[:N]


# speedup vs baseline: 1.1140x; 1.1140x over previous
import functools

import jax
import jax.numpy as jnp
from jax import lax
from jax.experimental import pallas as pl
from jax.experimental.pallas import tpu as pltpu
from jax.experimental.pallas import tpu_sc as plsc

N = 5000
TH = 0.5
L = 16
NS = 16
B = 128
N_PAD = 5120
NB = N_PAD // B
NCB = B // L
C = N_PAD // L
PAD = -1e30


def _nms_body(x1h, y1h, x2h, y2h, oh, outh,
              x1v, y1v, x2v, y2v, arv, alv, odv, stv, shv):
    sid = lax.axis_index("s")

    pltpu.sync_copy(x1h, x1v)
    pltpu.sync_copy(y1h, y1v)
    pltpu.sync_copy(x2h, x2v)
    pltpu.sync_copy(y2h, y2v)
    pltpu.sync_copy(oh, odv)

    def init_c(c, _):
        o = c * L
        w = jnp.maximum(x2v[pl.ds(o, L)] - x1v[pl.ds(o, L)], 0.0)
        h = jnp.maximum(y2v[pl.ds(o, L)] - y1v[pl.ds(o, L)], 0.0)
        arv[pl.ds(o, L)] = w * h
        alv[pl.ds(o, L)] = jnp.full((L,), 1.0, jnp.float32)
        return 0

    lax.fori_loop(0, N_PAD // L, init_c, 0)
    lanes = lax.iota(jnp.int32, L)

    def block_body(k, _):
        base = k * B
        j = jnp.remainder(sid - k * NCB, NS)

        @pl.when(j < NCB)
        def _():
            pltpu.sync_copy(alv.at[pl.ds(base + j * L, L)],
                            shv.at[pl.ds(j * L, L)])

        plsc.subcore_barrier()
        pltpu.sync_copy(shv, alv.at[pl.ds(base, B)])
        plsc.subcore_barrier()

        start = (k + 1) * NCB
        c0 = start + jnp.remainder(sid - start, NS)

        def chunk_body(cc, _):
            ci = k * NCB + cc
            oi = ci * L
            x1c = x1v[pl.ds(oi, L)]
            y1c = y1v[pl.ds(oi, L)]
            x2c = x2v[pl.ds(oi, L)]
            y2c = y2v[pl.ds(oi, L)]
            arc = arv[pl.ds(oi, L)]

            def lane_body(li, _):
                gi = oi + li
                a_i = alv[pl.ds(gi, L)][0]

                @pl.when(a_i > 0.0)
                def _():
                    liv = jnp.full((L,), li, jnp.int32)

                    def tk(vec):
                        return vec.at[liv].get(mode="promise_in_bounds")

                    x1i = tk(x1c)
                    y1i = tk(y1c)
                    x2i = tk(x2c)
                    y2i = tk(y2c)
                    ari = tk(arc)

                    def sup_chunk(c, extra=None):
                        o = c * L
                        ix1 = jnp.maximum(x1v[pl.ds(o, L)], x1i)
                        iy1 = jnp.maximum(y1v[pl.ds(o, L)], y1i)
                        ix2 = jnp.minimum(x2v[pl.ds(o, L)], x2i)
                        iy2 = jnp.minimum(y2v[pl.ds(o, L)], y2i)
                        inter = (jnp.maximum(ix2 - ix1, 0.0)
                                 * jnp.maximum(iy2 - iy1, 0.0))
                        union = arv[pl.ds(o, L)] + ari - inter
                        sup = inter > union * TH
                        if extra is not None:
                            sup = jnp.logical_and(sup, extra)
                        alv[pl.ds(o, L)] = jnp.where(sup, 0.0,
                                                     alv[pl.ds(o, L)])

                    sup_chunk(ci, lanes > li)

                    @plsc.parallel_loop(ci + 1, (k + 1) * NCB, unroll=4)
                    def _rest(c):
                        sup_chunk(c)

                    @plsc.parallel_loop(c0, C, step=NS, unroll=4)
                    def _tail(c):
                        sup_chunk(c)

                return 0

            lax.fori_loop(0, L, lane_body, 0)
            return 0

        lax.fori_loop(0, NCB, chunk_body, 0)
        return 0

    lax.fori_loop(0, NB, block_body, 0)

    def out_body(m, _):
        o = (m * NS + sid) * L
        stv[...] = jnp.where(alv[pl.ds(o, L)] > 0.0, odv[pl.ds(o, L)],
                             jnp.full((L,), -1, jnp.int32))
        pltpu.sync_copy(stv, outh.at[pl.ds(o, L)])
        return 0

    lax.fori_loop(0, C // NS, out_body, 0)


_nms_sc = functools.partial(
    pl.kernel,
    out_type=jax.ShapeDtypeStruct((N_PAD,), jnp.int32),
    mesh=plsc.VectorSubcoreMesh(core_axis_name="c", subcore_axis_name="s",
                                num_cores=1, num_subcores=NS),
    scratch_types=[
        pltpu.VMEM((N_PAD,), jnp.float32),
        pltpu.VMEM((N_PAD,), jnp.float32),
        pltpu.VMEM((N_PAD,), jnp.float32),
        pltpu.VMEM((N_PAD,), jnp.float32),
        pltpu.VMEM((N_PAD,), jnp.float32),
        pltpu.VMEM((N_PAD + L,), jnp.float32),
        pltpu.VMEM((N_PAD,), jnp.int32),
        pltpu.VMEM((L,), jnp.int32),
        pltpu.VMEM_SHARED((B,), jnp.float32),
    ],
)(_nms_body)


def kernel(boxes, scores, class_ids):
    max_c = boxes.max()
    offs = class_ids.astype(boxes.dtype) * (max_c + 1.0)
    b = boxes + offs[:, None]
    order = jnp.argsort(-scores)
    bs = b[order]
    padc = jnp.full((N_PAD - N,), PAD, jnp.float32)
    x1 = jnp.concatenate([bs[:, 0], padc])
    y1 = jnp.concatenate([bs[:, 1], padc])
    x2 = jnp.concatenate([bs[:, 2], padc])
    y2 = jnp.concatenate([bs[:, 3], padc])
    ordp = jnp.concatenate(
        [order.astype(jnp.int32), jnp.full((N_PAD - N,), -1, jnp.int32)])
    out = _nms_sc(x1, y1, x2, y2, ordp)
    return out[:N]
